# baseline (speedup 1.0000x reference)
import functools
import math

import jax
import jax.numpy as jnp
from jax.experimental import pallas as pl
from jax.experimental.pallas import tpu as pltpu


def _conv_kernel(x_ref, w_ref, o_ref, s_ref, wn_ref, *, h, w2, eps, gain):
    cg = x_ref.shape[1]
    k = w_ref.shape[1]
    inv_sqrt_k = 1.0 / math.sqrt(k)

    @pl.when(pl.program_id(0) == 0)
    def _():
        wf = w_ref[...].astype(jnp.float32)
        ssq = jnp.sum(wf * wf, axis=1, keepdims=True)
        scale = (gain * inv_sqrt_k) / (eps + jnp.sqrt(ssq) * inv_sqrt_k)
        wn_ref[...] = (wf * scale).astype(jnp.bfloat16)

    xb = x_ref[0]
    for di in (-1, 0, 1):
        for dj in (-1, 0, 1):
            of = di * w2 + dj
            if of == 0:
                s = xb
            elif of > 0:
                s = jnp.concatenate(
                    [xb[:, of:], jnp.zeros((cg, of), jnp.bfloat16)], axis=1)
            else:
                s = jnp.concatenate(
                    [jnp.zeros((cg, -of), jnp.bfloat16), xb[:, :of]], axis=1)
            tap = (di + 1) * 3 + (dj + 1)
            s_ref[tap * cg:(tap + 1) * cg, :] = s

    acc = jnp.dot(wn_ref[...], s_ref[...], preferred_element_type=jnp.float32)
    o_ref[0] = acc.astype(jnp.bfloat16)


def kernel(x, weight):
    n, cin, h, w = x.shape
    oc, cg, kh, kw = weight.shape
    khkw = kh * kw
    w2 = w + 2
    hw2 = h * w2

    wt = weight.transpose(0, 2, 3, 1).reshape(oc, khkw * cg)
    x16 = jnp.pad(x, ((0, 0), (0, 0), (0, 0), (1, 1))).astype(
        jnp.bfloat16).reshape(n, cin, hw2)

    kern = functools.partial(_conv_kernel, h=h, w2=w2, eps=1e-4, gain=1.0)
    flops = 2 * n * oc * hw2 * cg * khkw
    cost = pl.CostEstimate(
        flops=int(flops), transcendentals=0,
        bytes_accessed=int(x16.size * 2 + wt.size * 4 + n * oc * hw2 * 2))

    out = pl.pallas_call(
        kern,
        out_shape=jax.ShapeDtypeStruct((n, oc, hw2), jnp.bfloat16),
        grid=(n,),
        in_specs=[
            pl.BlockSpec((1, cin, hw2), lambda i: (i, 0, 0)),
            pl.BlockSpec((oc, khkw * cg), lambda i: (0, 0)),
        ],
        out_specs=pl.BlockSpec((1, oc, hw2), lambda i: (i, 0, 0)),
        scratch_shapes=[pltpu.VMEM((khkw * cg, hw2), jnp.bfloat16),
                        pltpu.VMEM((oc, khkw * cg), jnp.bfloat16)],
        compiler_params=pltpu.CompilerParams(
            dimension_semantics=("parallel",),
            vmem_limit_bytes=48 * 1024 * 1024),
        cost_estimate=cost,
    )(x16, wt)
    out = out.astype(jnp.float32).reshape(n, oc, h, w2)
    return out[:, :, :, 1:w + 1]


# speedup vs baseline: 2.0590x; 2.0590x over previous
import functools
import math

import jax
import jax.numpy as jnp
from jax.experimental import pallas as pl
from jax.experimental.pallas import tpu as pltpu


def _conv_kernel(x_ref, w_ref, o_ref, s_ref, wn_ref, *, h, w, eps, gain):
    cg = x_ref.shape[1]
    hw = h * w
    k = w_ref.shape[1]
    inv_sqrt_k = 1.0 / math.sqrt(k)

    @pl.when(pl.program_id(0) == 0)
    def _():
        wf = w_ref[...].astype(jnp.float32)
        ssq = jnp.sum(wf * wf, axis=1, keepdims=True)
        scale = (gain * inv_sqrt_k) / (eps + jnp.sqrt(ssq) * inv_sqrt_k)
        wn_ref[...] = (wf * scale).astype(jnp.bfloat16)

    xb = x_ref[0].astype(jnp.bfloat16)
    col = jax.lax.broadcasted_iota(jnp.int32, (1, hw), 1)
    col = (col & (w - 1)) if (w & (w - 1)) == 0 else (col % w)
    xm_l = jnp.where(col != w - 1, xb, jnp.bfloat16(0))
    xm_r = jnp.where(col != 0, xb, jnp.bfloat16(0))

    for di in (-1, 0, 1):
        for dj in (-1, 0, 1):
            src = xm_l if dj == -1 else (xm_r if dj == 1 else xb)
            of = di * w + dj
            if of == 0:
                s = src
            elif of > 0:
                s = jnp.concatenate(
                    [src[:, of:], jnp.zeros((cg, of), jnp.bfloat16)], axis=1)
            else:
                s = jnp.concatenate(
                    [jnp.zeros((cg, -of), jnp.bfloat16), src[:, :of]], axis=1)
            tap = (di + 1) * 3 + (dj + 1)
            s_ref[tap * cg:(tap + 1) * cg, :] = s

    acc = jnp.dot(wn_ref[...], s_ref[...], preferred_element_type=jnp.float32)
    o_ref[0] = acc.astype(jnp.bfloat16)


def kernel(x, weight):
    n, cin, h, w = x.shape
    oc, cg, kh, kw = weight.shape
    khkw = kh * kw
    hw = h * w

    wt = weight.transpose(0, 2, 3, 1).reshape(oc, khkw * cg)
    x16 = x.reshape(n, cin, hw)

    kern = functools.partial(_conv_kernel, h=h, w=w, eps=1e-4, gain=1.0)
    flops = 2 * n * oc * hw * cg * khkw
    cost = pl.CostEstimate(
        flops=int(flops), transcendentals=0,
        bytes_accessed=int(x16.size * 4 + wt.size * 4 + n * oc * hw * 2))

    out = pl.pallas_call(
        kern,
        out_shape=jax.ShapeDtypeStruct((n, oc, hw), jnp.bfloat16),
        grid=(n,),
        in_specs=[
            pl.BlockSpec((1, cin, hw), lambda i: (i, 0, 0)),
            pl.BlockSpec((oc, khkw * cg), lambda i: (0, 0)),
        ],
        out_specs=pl.BlockSpec((1, oc, hw), lambda i: (i, 0, 0)),
        scratch_shapes=[pltpu.VMEM((khkw * cg, hw), jnp.bfloat16),
                        pltpu.VMEM((oc, khkw * cg), jnp.bfloat16)],
        compiler_params=pltpu.CompilerParams(
            dimension_semantics=("parallel",),
            vmem_limit_bytes=48 * 1024 * 1024),
        cost_estimate=cost,
    )(x16, wt)
    return out.astype(jnp.float32).reshape(n, oc, h, w)
